# Initial kernel scaffold; baseline (speedup 1.0000x reference)
import jax
import jax.numpy as jnp
from jax.experimental import pallas as pl


def kernel(b, A, W1, b1, W2, b2, W3, b3, Wc1, bc1, Wc2, bc2, Wf1, bf1, Wf2, bf2, max_param, edge_index):
    raise NotImplementedError("write your pallas kernel here")


# speedup vs baseline: 11.9718x; 11.9718x over previous
import functools

import jax
import jax.numpy as jnp
from jax import lax
from jax.experimental import pallas as pl
from jax.experimental.pallas import tpu as pltpu

B = 16
P = 64
M = 32
N_DIM = 256
H = 64
K_IT = 10
E = 512


def _leaky(x):
    return jnp.where(x >= 0, x, 0.01 * x)


def _graph_kernel(src_ref, dst_ref, adj_ref, lap_ref, deg_ref):
    src = src_ref[...]
    dst = dst_ref[...]
    e_iota = lax.broadcasted_iota(jnp.int32, (B, E, P), 2)
    s_oh = (src[:, :, None] == e_iota).astype(jnp.float32)
    d_oh = (dst[:, :, None] == e_iota).astype(jnp.float32)
    c = lax.dot_general(d_oh, s_oh, (((1,), (1,)), ((0,), (0,))),
                        preferred_element_type=jnp.float32)
    ct = lax.dot_general(s_oh, d_oh, (((1,), (1,)), ((0,), (0,))),
                         preferred_element_type=jnp.float32)
    deg_d = jnp.sum(c, axis=2)
    deg_s = jnp.sum(ct, axis=2)
    ii = lax.broadcasted_iota(jnp.int32, (P, P), 0)
    jj = lax.broadcasted_iota(jnp.int32, (P, P), 1)
    eye = (ii == jj).astype(jnp.float32)[None]
    dinv = lax.rsqrt(deg_d + 1.0)
    adj_ref[...] = dinv[:, :, None] * dinv[:, None, :] * (c + eye)
    lap_ref[...] = eye * (deg_s + deg_d)[:, :, None] - c - ct
    deg_ref[...] = deg_s


def _graph_ops(src, dst):
    return pl.pallas_call(
        _graph_kernel,
        out_shape=[
            jax.ShapeDtypeStruct((B, P, P), jnp.float32),
            jax.ShapeDtypeStruct((B, P, P), jnp.float32),
            jax.ShapeDtypeStruct((B, P), jnp.float32),
        ],
    )(src, dst)


def _mlp_kernel(x_ref, w_ref, b_ref, o_ref, acc_ref, *, nk, act):
    k = pl.program_id(1)

    @pl.when(k == 0)
    def _():
        acc_ref[...] = jnp.zeros_like(acc_ref)

    acc_ref[...] += jnp.dot(x_ref[...], w_ref[...],
                            preferred_element_type=jnp.float32)

    @pl.when(k == nk - 1)
    def _():
        r = acc_ref[...] + b_ref[...]
        o_ref[...] = _leaky(r) if act else r


def _mlp_layer(x, w, bias, act, kb=2048, nb=512):
    mdim, kdim = x.shape
    ndim = w.shape[1]
    kb = min(kb, kdim)
    nk = kdim // kb
    grid = (ndim // nb, nk)
    return pl.pallas_call(
        functools.partial(_mlp_kernel, nk=nk, act=act),
        grid=grid,
        in_specs=[
            pl.BlockSpec((mdim, kb), lambda j, k: (0, k)),
            pl.BlockSpec((kb, nb), lambda j, k: (k, j)),
            pl.BlockSpec((1, nb), lambda j, k: (0, j)),
        ],
        out_specs=pl.BlockSpec((mdim, nb), lambda j, k: (0, j)),
        out_shape=jax.ShapeDtypeStruct((mdim, ndim), jnp.float32),
        scratch_shapes=[pltpu.VMEM((mdim, nb), jnp.float32)],
        compiler_params=pltpu.CompilerParams(
            dimension_semantics=("parallel", "arbitrary")),
    )(x, w, bias.reshape(1, ndim))


def _head_kernel(x_ref, adj_ref, wc1_ref, bc1_ref, wc2_ref, bc2_ref,
                 wf1_ref, bf1_ref, wf2_ref, bf2_ref, mp_ref, o_ref):
    x = x_ref[...]
    adj = adj_ref[...]
    xw = lax.dot_general(x, wc1_ref[...], (((2,), (0,)), ((), ())),
                         preferred_element_type=jnp.float32)
    h = lax.dot_general(adj, xw, (((2,), (1,)), ((0,), (0,))),
                        preferred_element_type=jnp.float32)
    h = _leaky(h + bc1_ref[...][None])
    hw = lax.dot_general(h, wc2_ref[...], (((2,), (0,)), ((), ())),
                         preferred_element_type=jnp.float32)
    h2 = lax.dot_general(adj, hw, (((2,), (1,)), ((0,), (0,))),
                         preferred_element_type=jnp.float32)
    h2 = _leaky(h2 + bc2_ref[...][None])
    pool = jnp.mean(h2, axis=1)
    f = _leaky(jnp.dot(pool, wf1_ref[...],
                       preferred_element_type=jnp.float32) + bf1_ref[...])
    g = jnp.dot(f, wf2_ref[...],
                preferred_element_type=jnp.float32) + bf2_ref[...]
    mp = mp_ref[...]
    acc = jnp.zeros((B, P * 4), jnp.float32)
    for k in range(K_IT):
        acc = acc + g[:, k * P * 4:(k + 1) * P * 4]
        o_ref[:, k * P * 4:(k + 1) * P * 4] = jax.nn.sigmoid(acc) * mp


def _head(x, adj, wc1, bc1, wc2, bc2, wf1, bf1, wf2, bf2, mp):
    return pl.pallas_call(
        _head_kernel,
        out_shape=jax.ShapeDtypeStruct((B, K_IT * P * 4), jnp.float32),
    )(x, adj, wc1, bc1.reshape(1, -1), wc2, bc2.reshape(1, -1),
      wf1, bf1.reshape(1, -1), wf2, bf2.reshape(1, -1), mp)


def _admm_kernel(a0_ref, bt_ref, y0_ref, u0_ref, d0_ref, lap_ref,
                 ha_ref, ht_ref, hr_ref, he_ref, sn_ref, o_ref,
                 ata_ref, atb_ref, y_ref, u_ref, d_ref):
    a0 = a0_ref[...]
    ata_ref[...] = lax.dot_general(a0, a0, (((1,), (1,)), ((0,), (0,))),
                                   preferred_element_type=jnp.float32)
    atb_ref[...] = lax.dot_general(bt_ref[...], a0,
                                   (((2,), (1,)), ((0,), (0,))),
                                   preferred_element_type=jnp.float32)
    y_ref[...] = y0_ref[...]
    u_ref[...] = u0_ref[...]
    d_ref[...] = d0_ref[...]
    sn = sn_ref[...][:, :, None]

    def step(k, _):
        al = jnp.reshape(ha_ref[pl.ds(k, 1)], (P, B))[:, :, None]
        ta = jnp.reshape(ht_ref[pl.ds(k, 1)], (P, B))[:, :, None]
        rh = jnp.reshape(hr_ref[pl.ds(k, 1)], (P, B))[:, :, None]
        et = jnp.reshape(he_ref[pl.ds(k, 1)], (P, B))[:, :, None]
        y = y_ref[...]
        atay = lax.dot_general(y, ata_ref[...], (((2,), (2,)), ((0,), (0,))),
                               preferred_element_type=jnp.float32)
        grad = (atay - atb_ref[...] + jnp.sign(y) * ta
                + u_ref[...] * sn + d_ref[...] * rh)
        y_next = y - al * grad
        for bb in range(B):
            yb = y_next[:, bb, :]
            lb = lap_ref[bb]
            d_ref[:, bb, :] = jnp.dot(lb, yb,
                                      preferred_element_type=jnp.float32)
        u_ref[...] = u_ref[...] + d_ref[...] * et
        y_ref[...] = y_next
        o_ref[pl.ds(k, 1)] = y_next[None]
        return 0

    lax.fori_loop(0, K_IT, step, 0)


def _admm(a0, bt, y0, u0, d0, lap, ha, ht, hr, he, sn):
    return pl.pallas_call(
        _admm_kernel,
        out_shape=jax.ShapeDtypeStruct((K_IT, P, B, N_DIM), jnp.float32),
        scratch_shapes=[
            pltpu.VMEM((P, N_DIM, N_DIM), jnp.float32),
            pltpu.VMEM((P, B, N_DIM), jnp.float32),
            pltpu.VMEM((P, B, N_DIM), jnp.float32),
            pltpu.VMEM((P, B, N_DIM), jnp.float32),
            pltpu.VMEM((P, B, N_DIM), jnp.float32),
        ],
    )(a0, bt, y0, u0, d0, lap, ha, ht, hr, he, sn)


def kernel(b, A, W1, b1, W2, b2, W3, b3, Wc1, bc1, Wc2, bc2,
           Wf1, bf1, Wf2, bf2, max_param, edge_index):
    src = edge_index[:, 0, :].astype(jnp.int32)
    dst = edge_index[:, 1, :].astype(jnp.int32)
    adj, lap, deg = _graph_ops(src, dst)

    x0 = b.reshape(B, P * M)
    x1 = _mlp_layer(x0, W1, b1, act=True)
    x2 = _mlp_layer(x1, W2, b2, act=True)
    x3 = _mlp_layer(x2, W3, b3, act=False)

    mp = jnp.tile(max_param.reshape(-1), P).reshape(1, P * 4)
    hyp_flat = _head(x3.reshape(B, P, 4 * H), adj, Wc1, bc1, Wc2, bc2,
                     Wf1, bf1, Wf2, bf2, mp)
    hyp4 = hyp_flat.reshape(B, K_IT, P, 4)
    ha = jnp.transpose(hyp4[..., 0], (1, 2, 0))
    ht = jnp.transpose(hyp4[..., 1], (1, 2, 0))
    hr = jnp.transpose(hyp4[..., 2], (1, 2, 0))
    he = jnp.transpose(hyp4[..., 3], (1, 2, 0))

    kk = jax.random.split(jax.random.key(1), 3)
    y0 = jax.random.normal(kk[0], (B, P, N_DIM, 1), dtype=jnp.float32) * 0.01
    u0 = jax.random.normal(kk[1], (B, P, N_DIM, 1), dtype=jnp.float32) * 0.01
    dl0 = jax.random.normal(kk[2], (B, P, N_DIM, 1), dtype=jnp.float32) * 0.01

    a0 = A[0]
    bt = jnp.transpose(b[..., 0], (1, 0, 2))
    y0t = jnp.transpose(y0[..., 0], (1, 0, 2))
    u0t = jnp.transpose(u0[..., 0], (1, 0, 2))
    d0t = jnp.transpose(dl0[..., 0], (1, 0, 2))
    snt = jnp.transpose(deg, (1, 0))

    ys = _admm(a0, bt, y0t, u0t, d0t, lap, ha, ht, hr, he, snt)
    return jnp.transpose(ys, (0, 2, 1, 3))[..., None]


# speedup vs baseline: 13.4470x; 1.1232x over previous
import functools

import jax
import jax.numpy as jnp
import numpy as np
from jax import lax
from jax.experimental import pallas as pl
from jax.experimental.pallas import tpu as pltpu

B = 16
P = 64
M = 32
N_DIM = 256
H = 64
K_IT = 10
E = 512


def _init_state():
    kk = jax.random.split(jax.random.key(1), 3)
    out = []
    for i in range(3):
        v = jax.random.normal(kk[i], (B, P, N_DIM, 1), dtype=jnp.float32)
        out.append(np.transpose(np.asarray(v)[..., 0], (1, 0, 2)) * 0.01)
    return out


_Y0, _U0, _D0 = _init_state()


def _leaky(x):
    return jnp.where(x >= 0, x, 0.01 * x)


def _graph_kernel(src_ref, dst_ref, adj_ref, lap_ref, deg_ref):
    src = src_ref[...]
    dst = dst_ref[...]
    e_iota = lax.broadcasted_iota(jnp.int32, (B, E, P), 2)
    s_oh = (src[:, :, None] == e_iota).astype(jnp.float32)
    d_oh = (dst[:, :, None] == e_iota).astype(jnp.float32)
    c = lax.dot_general(d_oh, s_oh, (((1,), (1,)), ((0,), (0,))),
                        preferred_element_type=jnp.float32)
    ct = lax.dot_general(s_oh, d_oh, (((1,), (1,)), ((0,), (0,))),
                         preferred_element_type=jnp.float32)
    deg_d = jnp.sum(c, axis=2)
    deg_s = jnp.sum(ct, axis=2)
    ii = lax.broadcasted_iota(jnp.int32, (P, P), 0)
    jj = lax.broadcasted_iota(jnp.int32, (P, P), 1)
    eye = (ii == jj).astype(jnp.float32)[None]
    dinv = lax.rsqrt(deg_d + 1.0)
    adj_ref[...] = dinv[:, :, None] * dinv[:, None, :] * (c + eye)
    lap_ref[...] = eye * (deg_s + deg_d)[:, :, None] - c - ct
    deg_ref[...] = deg_s


def _graph_ops(src, dst):
    return pl.pallas_call(
        _graph_kernel,
        out_shape=[
            jax.ShapeDtypeStruct((B, P, P), jnp.float32),
            jax.ShapeDtypeStruct((B, P, P), jnp.float32),
            jax.ShapeDtypeStruct((B, P), jnp.float32),
        ],
    )(src, dst)


def _mlp_kernel(x_ref, w_ref, b_ref, o_ref, acc_ref, *, nk, act):
    k = pl.program_id(1)

    @pl.when(k == 0)
    def _():
        acc_ref[...] = jnp.zeros_like(acc_ref)

    acc_ref[...] += jnp.dot(x_ref[...], w_ref[...],
                            preferred_element_type=jnp.float32)

    @pl.when(k == nk - 1)
    def _():
        r = acc_ref[...] + b_ref[...]
        o_ref[...] = _leaky(r) if act else r


def _mlp_layer(x, w, bias, act, kb=2048, nb=512):
    mdim, kdim = x.shape
    ndim = w.shape[1]
    kb = min(kb, kdim)
    nk = kdim // kb
    grid = (ndim // nb, nk)
    return pl.pallas_call(
        functools.partial(_mlp_kernel, nk=nk, act=act),
        grid=grid,
        in_specs=[
            pl.BlockSpec((mdim, kb), lambda j, k: (0, k)),
            pl.BlockSpec((kb, nb), lambda j, k: (k, j)),
            pl.BlockSpec((1, nb), lambda j, k: (0, j)),
        ],
        out_specs=pl.BlockSpec((mdim, nb), lambda j, k: (0, j)),
        out_shape=jax.ShapeDtypeStruct((mdim, ndim), jnp.float32),
        scratch_shapes=[pltpu.VMEM((mdim, nb), jnp.float32)],
        compiler_params=pltpu.CompilerParams(
            dimension_semantics=("parallel", "arbitrary")),
    )(x, w, bias.reshape(1, ndim))


def _head_kernel(x_ref, adj_ref, wc1_ref, bc1_ref, wc2_ref, bc2_ref,
                 wf1_ref, bf1_ref, wf2_ref, bf2_ref, mp_ref, o_ref):
    x = x_ref[...]
    adj = adj_ref[...]
    xw = lax.dot_general(x, wc1_ref[...], (((2,), (0,)), ((), ())),
                         preferred_element_type=jnp.float32)
    h = lax.dot_general(adj, xw, (((2,), (1,)), ((0,), (0,))),
                        preferred_element_type=jnp.float32)
    h = _leaky(h + bc1_ref[...][None])
    hw = lax.dot_general(h, wc2_ref[...], (((2,), (0,)), ((), ())),
                         preferred_element_type=jnp.float32)
    h2 = lax.dot_general(adj, hw, (((2,), (1,)), ((0,), (0,))),
                         preferred_element_type=jnp.float32)
    h2 = _leaky(h2 + bc2_ref[...][None])
    pool = jnp.mean(h2, axis=1)
    f = _leaky(jnp.dot(pool, wf1_ref[...],
                       preferred_element_type=jnp.float32) + bf1_ref[...])
    g = jnp.dot(f, wf2_ref[...],
                preferred_element_type=jnp.float32) + bf2_ref[...]
    mp = mp_ref[...]
    acc = jnp.zeros((B, P * 4), jnp.float32)
    for k in range(K_IT):
        acc = acc + g[:, k * P * 4:(k + 1) * P * 4]
        o_ref[:, k * P * 4:(k + 1) * P * 4] = jax.nn.sigmoid(acc) * mp


def _head(x, adj, wc1, bc1, wc2, bc2, wf1, bf1, wf2, bf2, mp):
    return pl.pallas_call(
        _head_kernel,
        out_shape=jax.ShapeDtypeStruct((B, K_IT * P * 4), jnp.float32),
    )(x, adj, wc1, bc1.reshape(1, -1), wc2, bc2.reshape(1, -1),
      wf1, bf1.reshape(1, -1), wf2, bf2.reshape(1, -1), mp)


def _admm_kernel(a0_ref, bt_ref, y0_ref, u0_ref, d0_ref, lap_ref,
                 ha_ref, ht_ref, hr_ref, he_ref, sn_ref, o_ref,
                 atb_ref, y_ref, u_ref, d_ref):
    a0 = a0_ref[...]
    atb_ref[...] = lax.dot_general(bt_ref[...], a0,
                                   (((2,), (1,)), ((0,), (0,))),
                                   preferred_element_type=jnp.float32)
    y_ref[...] = y0_ref[...]
    u_ref[...] = u0_ref[...]
    d_ref[...] = d0_ref[...]
    sn = sn_ref[...][:, :, None]

    def step(k, _):
        al = jnp.reshape(ha_ref[pl.ds(k, 1)], (P, B))[:, :, None]
        ta = jnp.reshape(ht_ref[pl.ds(k, 1)], (P, B))[:, :, None]
        rh = jnp.reshape(hr_ref[pl.ds(k, 1)], (P, B))[:, :, None]
        et = jnp.reshape(he_ref[pl.ds(k, 1)], (P, B))[:, :, None]
        y = y_ref[...]
        ay = lax.dot_general(y, a0, (((2,), (2,)), ((0,), (0,))),
                             preferred_element_type=jnp.float32)
        atay = lax.dot_general(ay, a0, (((2,), (1,)), ((0,), (0,))),
                               preferred_element_type=jnp.float32)
        grad = (atay - atb_ref[...] + jnp.sign(y) * ta
                + u_ref[...] * sn + d_ref[...] * rh)
        y_next = y - al * grad
        for bb in range(B):
            yb = y_next[:, bb, :]
            lb = lap_ref[bb]
            db = jnp.dot(lb, yb, preferred_element_type=jnp.float32)
            d_ref[:, bb, :] = db
            o_ref[pl.ds(k, 1), bb] = yb[None]
        u_ref[...] = u_ref[...] + d_ref[...] * et
        y_ref[...] = y_next
        return 0

    lax.fori_loop(0, K_IT, step, 0)


def _admm(a0, bt, y0, u0, d0, lap, ha, ht, hr, he, sn):
    return pl.pallas_call(
        _admm_kernel,
        out_shape=jax.ShapeDtypeStruct((K_IT, B, P, N_DIM), jnp.float32),
        scratch_shapes=[
            pltpu.VMEM((P, B, N_DIM), jnp.float32),
            pltpu.VMEM((P, B, N_DIM), jnp.float32),
            pltpu.VMEM((P, B, N_DIM), jnp.float32),
            pltpu.VMEM((P, B, N_DIM), jnp.float32),
        ],
    )(a0, bt, y0, u0, d0, lap, ha, ht, hr, he, sn)


def kernel(b, A, W1, b1, W2, b2, W3, b3, Wc1, bc1, Wc2, bc2,
           Wf1, bf1, Wf2, bf2, max_param, edge_index):
    src = edge_index[:, 0, :].astype(jnp.int32)
    dst = edge_index[:, 1, :].astype(jnp.int32)
    adj, lap, deg = _graph_ops(src, dst)

    x0 = b.reshape(B, P * M)
    x1 = _mlp_layer(x0, W1, b1, act=True)
    x2 = _mlp_layer(x1, W2, b2, act=True)
    x3 = _mlp_layer(x2, W3, b3, act=False)

    mp = jnp.tile(max_param.reshape(-1), P).reshape(1, P * 4)
    hyp_flat = _head(x3.reshape(B, P, 4 * H), adj, Wc1, bc1, Wc2, bc2,
                     Wf1, bf1, Wf2, bf2, mp)
    hyp4 = hyp_flat.reshape(B, K_IT, P, 4)
    ha = jnp.transpose(hyp4[..., 0], (1, 2, 0))
    ht = jnp.transpose(hyp4[..., 1], (1, 2, 0))
    hr = jnp.transpose(hyp4[..., 2], (1, 2, 0))
    he = jnp.transpose(hyp4[..., 3], (1, 2, 0))

    a0 = A[0]
    bt = jnp.transpose(b[..., 0], (1, 0, 2))
    snt = jnp.transpose(deg, (1, 0))

    ys = _admm(a0, bt, jnp.asarray(_Y0), jnp.asarray(_U0), jnp.asarray(_D0),
               lap, ha, ht, hr, he, snt)
    return ys[..., None]


# speedup vs baseline: 13.5178x; 1.0053x over previous
import functools

import jax
import jax.numpy as jnp
import numpy as np
from jax import lax
from jax.experimental import pallas as pl
from jax.experimental.pallas import tpu as pltpu

B = 16
P = 64
M = 32
N_DIM = 256
H = 64
K_IT = 10
E = 512


def _threefry2x32(k0, k1, x0, x1):
    rot = (13, 15, 26, 6, 17, 29, 16, 24)
    k0 = np.uint32(k0)
    k1 = np.uint32(k1)
    ks = (k0, k1, np.uint32(k0 ^ k1 ^ np.uint32(0x1BD11BDA)))
    x0 = (x0 + ks[0]).astype(np.uint32)
    x1 = (x1 + ks[1]).astype(np.uint32)
    for i in range(5):
        for r in rot[(i % 2) * 4:(i % 2) * 4 + 4]:
            x0 = (x0 + x1).astype(np.uint32)
            x1 = ((x1 << np.uint32(r)) | (x1 >> np.uint32(32 - r))).astype(np.uint32)
            x1 = (x1 ^ x0).astype(np.uint32)
        x0 = (x0 + ks[(i + 1) % 3]).astype(np.uint32)
        x1 = (x1 + ks[(i + 2) % 3] + np.uint32(i + 1)).astype(np.uint32)
    return x0, x1


def _erfinv64(x):
    x = x.astype(np.float64)
    w = -np.log1p(-x * x)
    p_lo = np.full_like(w, 2.81022636e-08)
    wl = w - 2.5
    for c in (3.43273939e-07, -3.5233877e-06, -4.39150654e-06, 0.00021858087,
              -0.00125372503, -0.00417768164, 0.246640727, 1.50140941):
        p_lo = c + p_lo * wl
    ws = np.sqrt(np.maximum(w, 5.0)) - 3.0
    p_hi = np.full_like(w, -0.000200214257)
    for c in (0.000100950558, 0.00134934322, -0.00367342844, 0.00573950773,
              -0.0076224613, 0.00943887047, 1.00167406, 2.83297682):
        p_hi = c + p_hi * ws
    return np.where(w < 5.0, p_lo, p_hi) * x


def _init_state():
    n = B * P * N_DIM
    with np.errstate(over="ignore"):
        s1, s2 = _threefry2x32(0, 1, np.zeros(3, np.uint32),
                               np.arange(3, dtype=np.uint32))
        out = []
        for i in range(3):
            b1, b2 = _threefry2x32(s1[i], s2[i], np.zeros(n, np.uint32),
                                   np.arange(n, dtype=np.uint32))
            bits = (b1 ^ b2).astype(np.uint32)
            f = ((bits >> np.uint32(9)) | np.uint32(0x3F800000)).view(np.float32)
            f = f - np.float32(1.0)
            lo = np.float32(np.nextafter(np.float32(-1.0), np.float32(0.0)))
            u = np.maximum(lo, (f * (np.float32(1.0) - lo) + lo).astype(np.float32))
            v = (np.sqrt(2.0) * _erfinv64(u)).astype(np.float32)
            v = v.reshape(B, P, N_DIM)
            out.append(np.transpose(v, (1, 0, 2)) * np.float32(0.01))
    return out


_Y0, _U0, _D0 = _init_state()


def _leaky(x):
    return jnp.where(x >= 0, x, 0.01 * x)


def _graph_kernel(src_ref, dst_ref, adj_ref, lap_ref, deg_ref):
    src = src_ref[...]
    dst = dst_ref[...]
    e_iota = lax.broadcasted_iota(jnp.int32, (B, E, P), 2)
    s_oh = (src[:, :, None] == e_iota).astype(jnp.float32)
    d_oh = (dst[:, :, None] == e_iota).astype(jnp.float32)
    c = lax.dot_general(d_oh, s_oh, (((1,), (1,)), ((0,), (0,))),
                        preferred_element_type=jnp.float32)
    ct = lax.dot_general(s_oh, d_oh, (((1,), (1,)), ((0,), (0,))),
                         preferred_element_type=jnp.float32)
    deg_d = jnp.sum(c, axis=2)
    deg_s = jnp.sum(ct, axis=2)
    ii = lax.broadcasted_iota(jnp.int32, (P, P), 0)
    jj = lax.broadcasted_iota(jnp.int32, (P, P), 1)
    eye = (ii == jj).astype(jnp.float32)[None]
    dinv = lax.rsqrt(deg_d + 1.0)
    adj_ref[...] = dinv[:, :, None] * dinv[:, None, :] * (c + eye)
    lap_ref[...] = eye * (deg_s + deg_d)[:, :, None] - c - ct
    deg_ref[...] = deg_s


def _graph_ops(src, dst):
    return pl.pallas_call(
        _graph_kernel,
        out_shape=[
            jax.ShapeDtypeStruct((B, P, P), jnp.float32),
            jax.ShapeDtypeStruct((B, P, P), jnp.float32),
            jax.ShapeDtypeStruct((B, P), jnp.float32),
        ],
    )(src, dst)


def _mlp_kernel(x_ref, w_ref, b_ref, o_ref, acc_ref, *, nk, act):
    k = pl.program_id(1)

    @pl.when(k == 0)
    def _():
        acc_ref[...] = jnp.zeros_like(acc_ref)

    acc_ref[...] += jnp.dot(x_ref[...], w_ref[...],
                            preferred_element_type=jnp.float32)

    @pl.when(k == nk - 1)
    def _():
        r = acc_ref[...] + b_ref[...]
        o_ref[...] = _leaky(r) if act else r


def _mlp_layer(x, w, bias, act, kb=2048, nb=512):
    mdim, kdim = x.shape
    ndim = w.shape[1]
    kb = min(kb, kdim)
    nk = kdim // kb
    grid = (ndim // nb, nk)
    return pl.pallas_call(
        functools.partial(_mlp_kernel, nk=nk, act=act),
        grid=grid,
        in_specs=[
            pl.BlockSpec((mdim, kb), lambda j, k: (0, k)),
            pl.BlockSpec((kb, nb), lambda j, k: (k, j)),
            pl.BlockSpec((1, nb), lambda j, k: (0, j)),
        ],
        out_specs=pl.BlockSpec((mdim, nb), lambda j, k: (0, j)),
        out_shape=jax.ShapeDtypeStruct((mdim, ndim), jnp.float32),
        scratch_shapes=[pltpu.VMEM((mdim, nb), jnp.float32)],
        compiler_params=pltpu.CompilerParams(
            dimension_semantics=("parallel", "arbitrary")),
    )(x, w, bias.reshape(1, ndim))


def _head_kernel(x_ref, adj_ref, wc1_ref, bc1_ref, wc2_ref, bc2_ref,
                 wf1_ref, bf1_ref, wf2_ref, bf2_ref, mp_ref, o_ref):
    x = x_ref[...]
    adj = adj_ref[...]
    xw = lax.dot_general(x, wc1_ref[...], (((2,), (0,)), ((), ())),
                         preferred_element_type=jnp.float32)
    h = lax.dot_general(adj, xw, (((2,), (1,)), ((0,), (0,))),
                        preferred_element_type=jnp.float32)
    h = _leaky(h + bc1_ref[...][None])
    hw = lax.dot_general(h, wc2_ref[...], (((2,), (0,)), ((), ())),
                         preferred_element_type=jnp.float32)
    h2 = lax.dot_general(adj, hw, (((2,), (1,)), ((0,), (0,))),
                         preferred_element_type=jnp.float32)
    h2 = _leaky(h2 + bc2_ref[...][None])
    pool = jnp.mean(h2, axis=1)
    f = _leaky(jnp.dot(pool, wf1_ref[...],
                       preferred_element_type=jnp.float32) + bf1_ref[...])
    g = jnp.dot(f, wf2_ref[...],
                preferred_element_type=jnp.float32) + bf2_ref[...]
    mp = mp_ref[...]
    acc = jnp.zeros((B, P * 4), jnp.float32)
    for k in range(K_IT):
        acc = acc + g[:, k * P * 4:(k + 1) * P * 4]
        o_ref[:, k * P * 4:(k + 1) * P * 4] = jax.nn.sigmoid(acc) * mp


def _head(x, adj, wc1, bc1, wc2, bc2, wf1, bf1, wf2, bf2, mp):
    return pl.pallas_call(
        _head_kernel,
        out_shape=jax.ShapeDtypeStruct((B, K_IT * P * 4), jnp.float32),
    )(x, adj, wc1, bc1.reshape(1, -1), wc2, bc2.reshape(1, -1),
      wf1, bf1.reshape(1, -1), wf2, bf2.reshape(1, -1), mp)


def _admm_kernel(a0_ref, bt_ref, y0_ref, u0_ref, d0_ref, lap_ref,
                 ha_ref, ht_ref, hr_ref, he_ref, sn_ref, o_ref,
                 atb_ref, y_ref, u_ref, d_ref):
    a0 = a0_ref[...]
    atb_ref[...] = lax.dot_general(bt_ref[...], a0,
                                   (((2,), (1,)), ((0,), (0,))),
                                   preferred_element_type=jnp.float32)
    y_ref[...] = y0_ref[...]
    u_ref[...] = u0_ref[...]
    d_ref[...] = d0_ref[...]
    sn = sn_ref[...][:, :, None]

    def step(k, _):
        al = jnp.reshape(ha_ref[pl.ds(k, 1)], (P, B))[:, :, None]
        ta = jnp.reshape(ht_ref[pl.ds(k, 1)], (P, B))[:, :, None]
        rh = jnp.reshape(hr_ref[pl.ds(k, 1)], (P, B))[:, :, None]
        et = jnp.reshape(he_ref[pl.ds(k, 1)], (P, B))[:, :, None]
        y = y_ref[...]
        ay = lax.dot_general(y, a0, (((2,), (2,)), ((0,), (0,))),
                             preferred_element_type=jnp.float32)
        atay = lax.dot_general(ay, a0, (((2,), (1,)), ((0,), (0,))),
                               preferred_element_type=jnp.float32)
        grad = (atay - atb_ref[...] + jnp.sign(y) * ta
                + u_ref[...] * sn + d_ref[...] * rh)
        y_next = y - al * grad
        for bb in range(B):
            yb = y_next[:, bb, :]
            lb = lap_ref[bb]
            db = jnp.dot(lb, yb, preferred_element_type=jnp.float32)
            d_ref[:, bb, :] = db
            o_ref[pl.ds(k, 1), bb] = yb[None]
        u_ref[...] = u_ref[...] + d_ref[...] * et
        y_ref[...] = y_next
        return 0

    lax.fori_loop(0, K_IT, step, 0)


def _admm(a0, bt, y0, u0, d0, lap, ha, ht, hr, he, sn):
    return pl.pallas_call(
        _admm_kernel,
        out_shape=jax.ShapeDtypeStruct((K_IT, B, P, N_DIM), jnp.float32),
        scratch_shapes=[
            pltpu.VMEM((P, B, N_DIM), jnp.float32),
            pltpu.VMEM((P, B, N_DIM), jnp.float32),
            pltpu.VMEM((P, B, N_DIM), jnp.float32),
            pltpu.VMEM((P, B, N_DIM), jnp.float32),
        ],
    )(a0, bt, y0, u0, d0, lap, ha, ht, hr, he, sn)


def kernel(b, A, W1, b1, W2, b2, W3, b3, Wc1, bc1, Wc2, bc2,
           Wf1, bf1, Wf2, bf2, max_param, edge_index):
    src = edge_index[:, 0, :].astype(jnp.int32)
    dst = edge_index[:, 1, :].astype(jnp.int32)
    adj, lap, deg = _graph_ops(src, dst)

    x0 = b.reshape(B, P * M)
    x1 = _mlp_layer(x0, W1, b1, act=True)
    x2 = _mlp_layer(x1, W2, b2, act=True)
    x3 = _mlp_layer(x2, W3, b3, act=False)

    mp = jnp.tile(max_param.reshape(-1), P).reshape(1, P * 4)
    hyp_flat = _head(x3.reshape(B, P, 4 * H), adj, Wc1, bc1, Wc2, bc2,
                     Wf1, bf1, Wf2, bf2, mp)
    hyp4 = hyp_flat.reshape(B, K_IT, P, 4)
    ha = jnp.transpose(hyp4[..., 0], (1, 2, 0))
    ht = jnp.transpose(hyp4[..., 1], (1, 2, 0))
    hr = jnp.transpose(hyp4[..., 2], (1, 2, 0))
    he = jnp.transpose(hyp4[..., 3], (1, 2, 0))

    a0 = A[0]
    bt = jnp.transpose(b[..., 0], (1, 0, 2))
    snt = jnp.transpose(deg, (1, 0))

    ys = _admm(a0, bt, jnp.asarray(_Y0), jnp.asarray(_U0), jnp.asarray(_D0),
               lap, ha, ht, hr, he, snt)
    return ys[..., None]


# speedup vs baseline: 14.2532x; 1.0544x over previous
import functools

import jax
import jax.numpy as jnp
import numpy as np
from jax import lax
from jax.experimental import pallas as pl
from jax.experimental.pallas import tpu as pltpu

B = 16
P = 64
M = 32
N_DIM = 256
H = 64
K_IT = 10
E = 512


def _threefry2x32(k0, k1, x0, x1):
    rot = (13, 15, 26, 6, 17, 29, 16, 24)
    k0 = np.uint32(k0)
    k1 = np.uint32(k1)
    ks = (k0, k1, np.uint32(k0 ^ k1 ^ np.uint32(0x1BD11BDA)))
    x0 = (x0 + ks[0]).astype(np.uint32)
    x1 = (x1 + ks[1]).astype(np.uint32)
    for i in range(5):
        for r in rot[(i % 2) * 4:(i % 2) * 4 + 4]:
            x0 = (x0 + x1).astype(np.uint32)
            x1 = ((x1 << np.uint32(r)) | (x1 >> np.uint32(32 - r))).astype(np.uint32)
            x1 = (x1 ^ x0).astype(np.uint32)
        x0 = (x0 + ks[(i + 1) % 3]).astype(np.uint32)
        x1 = (x1 + ks[(i + 2) % 3] + np.uint32(i + 1)).astype(np.uint32)
    return x0, x1


def _erfinv64(x):
    x = x.astype(np.float64)
    w = -np.log1p(-x * x)
    p_lo = np.full_like(w, 2.81022636e-08)
    wl = w - 2.5
    for c in (3.43273939e-07, -3.5233877e-06, -4.39150654e-06, 0.00021858087,
              -0.00125372503, -0.00417768164, 0.246640727, 1.50140941):
        p_lo = c + p_lo * wl
    ws = np.sqrt(np.maximum(w, 5.0)) - 3.0
    p_hi = np.full_like(w, -0.000200214257)
    for c in (0.000100950558, 0.00134934322, -0.00367342844, 0.00573950773,
              -0.0076224613, 0.00943887047, 1.00167406, 2.83297682):
        p_hi = c + p_hi * ws
    return np.where(w < 5.0, p_lo, p_hi) * x


def _init_state():
    n = B * P * N_DIM
    with np.errstate(over="ignore"):
        s1, s2 = _threefry2x32(0, 1, np.zeros(3, np.uint32),
                               np.arange(3, dtype=np.uint32))
        out = []
        for i in range(3):
            b1, b2 = _threefry2x32(s1[i], s2[i], np.zeros(n, np.uint32),
                                   np.arange(n, dtype=np.uint32))
            bits = (b1 ^ b2).astype(np.uint32)
            f = ((bits >> np.uint32(9)) | np.uint32(0x3F800000)).view(np.float32)
            f = f - np.float32(1.0)
            lo = np.float32(np.nextafter(np.float32(-1.0), np.float32(0.0)))
            u = np.maximum(lo, (f * (np.float32(1.0) - lo) + lo).astype(np.float32))
            v = (np.sqrt(2.0) * _erfinv64(u)).astype(np.float32)
            v = v.reshape(B, P, N_DIM)
            out.append(np.transpose(v, (1, 0, 2)) * np.float32(0.01))
    return out


_Y0, _U0, _D0 = _init_state()


def _leaky(x):
    return jnp.where(x >= 0, x, 0.01 * x)


def _graph_kernel(src_ref, dst_ref, adj_ref, lap_ref, deg_ref):
    src = src_ref[...]
    dst = dst_ref[...]
    e_iota = lax.broadcasted_iota(jnp.int32, (B, E, P), 2)
    s_oh = (src[:, :, None] == e_iota).astype(jnp.float32)
    d_oh = (dst[:, :, None] == e_iota).astype(jnp.float32)
    c = lax.dot_general(d_oh, s_oh, (((1,), (1,)), ((0,), (0,))),
                        preferred_element_type=jnp.float32)
    ct = lax.dot_general(s_oh, d_oh, (((1,), (1,)), ((0,), (0,))),
                         preferred_element_type=jnp.float32)
    deg_d = jnp.sum(c, axis=2)
    deg_s = jnp.sum(ct, axis=2)
    ii = lax.broadcasted_iota(jnp.int32, (P, P), 0)
    jj = lax.broadcasted_iota(jnp.int32, (P, P), 1)
    eye = (ii == jj).astype(jnp.float32)[None]
    dinv = lax.rsqrt(deg_d + 1.0)
    adj_ref[...] = dinv[:, :, None] * dinv[:, None, :] * (c + eye)
    lap_ref[...] = eye * (deg_s + deg_d)[:, :, None] - c - ct
    deg_ref[...] = deg_s


def _graph_ops(src, dst):
    return pl.pallas_call(
        _graph_kernel,
        out_shape=[
            jax.ShapeDtypeStruct((B, P, P), jnp.float32),
            jax.ShapeDtypeStruct((B, P, P), jnp.float32),
            jax.ShapeDtypeStruct((B, P), jnp.float32),
        ],
    )(src, dst)


def _mlp_kernel(x_ref, w_ref, b_ref, o_ref, acc_ref, *, nk, act):
    k = pl.program_id(1)

    @pl.when(k == 0)
    def _():
        acc_ref[...] = jnp.zeros_like(acc_ref)

    acc_ref[...] += jnp.dot(x_ref[...], w_ref[...],
                            preferred_element_type=jnp.float32)

    @pl.when(k == nk - 1)
    def _():
        r = acc_ref[...] + b_ref[...]
        o_ref[...] = _leaky(r) if act else r


def _mlp_layer(x, w, bias, act, kb=2048, nb=1024):
    mdim, kdim = x.shape
    ndim = w.shape[1]
    kb = min(kb, kdim)
    nk = kdim // kb
    grid = (ndim // nb, nk)
    return pl.pallas_call(
        functools.partial(_mlp_kernel, nk=nk, act=act),
        grid=grid,
        in_specs=[
            pl.BlockSpec((mdim, kb), lambda j, k: (0, k)),
            pl.BlockSpec((kb, nb), lambda j, k: (k, j)),
            pl.BlockSpec((1, nb), lambda j, k: (0, j)),
        ],
        out_specs=pl.BlockSpec((mdim, nb), lambda j, k: (0, j)),
        out_shape=jax.ShapeDtypeStruct((mdim, ndim), jnp.float32),
        scratch_shapes=[pltpu.VMEM((mdim, nb), jnp.float32)],
        compiler_params=pltpu.CompilerParams(
            dimension_semantics=("parallel", "arbitrary")),
    )(x, w, bias.reshape(1, ndim))


def _head_kernel(x_ref, adj_ref, wc1_ref, bc1_ref, wc2_ref, bc2_ref,
                 wf1_ref, bf1_ref, wf2_ref, bf2_ref, mp_ref, o_ref):
    x = x_ref[...]
    adj = adj_ref[...]
    xw = lax.dot_general(x, wc1_ref[...], (((2,), (0,)), ((), ())),
                         preferred_element_type=jnp.float32)
    h = lax.dot_general(adj, xw, (((2,), (1,)), ((0,), (0,))),
                        preferred_element_type=jnp.float32)
    h = _leaky(h + bc1_ref[...][None])
    hw = lax.dot_general(h, wc2_ref[...], (((2,), (0,)), ((), ())),
                         preferred_element_type=jnp.float32)
    h2 = lax.dot_general(adj, hw, (((2,), (1,)), ((0,), (0,))),
                         preferred_element_type=jnp.float32)
    h2 = _leaky(h2 + bc2_ref[...][None])
    pool = jnp.mean(h2, axis=1)
    f = _leaky(jnp.dot(pool, wf1_ref[...],
                       preferred_element_type=jnp.float32) + bf1_ref[...])
    g = jnp.dot(f, wf2_ref[...],
                preferred_element_type=jnp.float32) + bf2_ref[...]
    mp = mp_ref[...]
    acc = jnp.zeros((B, P * 4), jnp.float32)
    for k in range(K_IT):
        acc = acc + g[:, k * P * 4:(k + 1) * P * 4]
        o_ref[:, k * P * 4:(k + 1) * P * 4] = jax.nn.sigmoid(acc) * mp


def _head(x, adj, wc1, bc1, wc2, bc2, wf1, bf1, wf2, bf2, mp):
    return pl.pallas_call(
        _head_kernel,
        out_shape=jax.ShapeDtypeStruct((B, K_IT * P * 4), jnp.float32),
    )(x, adj, wc1, bc1.reshape(1, -1), wc2, bc2.reshape(1, -1),
      wf1, bf1.reshape(1, -1), wf2, bf2.reshape(1, -1), mp)


def _admm_kernel(a0_ref, bt_ref, y0_ref, u0_ref, d0_ref, lap_ref,
                 ha_ref, ht_ref, hr_ref, he_ref, sn_ref, o_ref,
                 atb_ref, y_ref, u_ref, d_ref):
    a0 = a0_ref[...]
    atb_ref[...] = lax.dot_general(bt_ref[...], a0,
                                   (((2,), (1,)), ((0,), (0,))),
                                   preferred_element_type=jnp.float32)
    y_ref[...] = y0_ref[...]
    u_ref[...] = u0_ref[...]
    d_ref[...] = d0_ref[...]
    sn = sn_ref[...][:, :, None]

    def step(k, _):
        al = jnp.reshape(ha_ref[pl.ds(k, 1)], (P, B))[:, :, None]
        ta = jnp.reshape(ht_ref[pl.ds(k, 1)], (P, B))[:, :, None]
        rh = jnp.reshape(hr_ref[pl.ds(k, 1)], (P, B))[:, :, None]
        et = jnp.reshape(he_ref[pl.ds(k, 1)], (P, B))[:, :, None]
        y = y_ref[...]
        ay = lax.dot_general(y, a0, (((2,), (2,)), ((0,), (0,))),
                             preferred_element_type=jnp.float32)
        atay = lax.dot_general(ay, a0, (((2,), (1,)), ((0,), (0,))),
                               preferred_element_type=jnp.float32)
        grad = (atay - atb_ref[...] + jnp.sign(y) * ta
                + u_ref[...] * sn + d_ref[...] * rh)
        y_next = y - al * grad
        for bb in range(B):
            yb = y_next[:, bb, :]
            lb = lap_ref[bb]
            db = jnp.dot(lb, yb, preferred_element_type=jnp.float32)
            d_ref[:, bb, :] = db
            o_ref[pl.ds(k, 1), bb] = yb[None]
        u_ref[...] = u_ref[...] + d_ref[...] * et
        y_ref[...] = y_next
        return 0

    lax.fori_loop(0, K_IT, step, 0)


def _admm(a0, bt, y0, u0, d0, lap, ha, ht, hr, he, sn):
    return pl.pallas_call(
        _admm_kernel,
        out_shape=jax.ShapeDtypeStruct((K_IT, B, P, N_DIM), jnp.float32),
        scratch_shapes=[
            pltpu.VMEM((P, B, N_DIM), jnp.float32),
            pltpu.VMEM((P, B, N_DIM), jnp.float32),
            pltpu.VMEM((P, B, N_DIM), jnp.float32),
            pltpu.VMEM((P, B, N_DIM), jnp.float32),
        ],
    )(a0, bt, y0, u0, d0, lap, ha, ht, hr, he, sn)


def kernel(b, A, W1, b1, W2, b2, W3, b3, Wc1, bc1, Wc2, bc2,
           Wf1, bf1, Wf2, bf2, max_param, edge_index):
    src = edge_index[:, 0, :].astype(jnp.int32)
    dst = edge_index[:, 1, :].astype(jnp.int32)
    adj, lap, deg = _graph_ops(src, dst)

    x0 = b.reshape(B, P * M)
    x1 = _mlp_layer(x0, W1, b1, act=True)
    x2 = _mlp_layer(x1, W2, b2, act=True)
    x3 = _mlp_layer(x2, W3, b3, act=False)

    mp = jnp.tile(max_param.reshape(-1), P).reshape(1, P * 4)
    hyp_flat = _head(x3.reshape(B, P, 4 * H), adj, Wc1, bc1, Wc2, bc2,
                     Wf1, bf1, Wf2, bf2, mp)
    hyp4 = hyp_flat.reshape(B, K_IT, P, 4)
    ha = jnp.transpose(hyp4[..., 0], (1, 2, 0))
    ht = jnp.transpose(hyp4[..., 1], (1, 2, 0))
    hr = jnp.transpose(hyp4[..., 2], (1, 2, 0))
    he = jnp.transpose(hyp4[..., 3], (1, 2, 0))

    a0 = A[0]
    bt = jnp.transpose(b[..., 0], (1, 0, 2))
    snt = jnp.transpose(deg, (1, 0))

    ys = _admm(a0, bt, jnp.asarray(_Y0), jnp.asarray(_U0), jnp.asarray(_D0),
               lap, ha, ht, hr, he, snt)
    return ys[..., None]


# speedup vs baseline: 14.3785x; 1.0088x over previous
import functools

import jax
import jax.numpy as jnp
import numpy as np
from jax import lax
from jax.experimental import pallas as pl
from jax.experimental.pallas import tpu as pltpu

B = 16
P = 64
M = 32
N_DIM = 256
H = 64
K_IT = 10
E = 512


def _threefry2x32(k0, k1, x0, x1):
    rot = (13, 15, 26, 6, 17, 29, 16, 24)
    k0 = np.uint32(k0)
    k1 = np.uint32(k1)
    ks = (k0, k1, np.uint32(k0 ^ k1 ^ np.uint32(0x1BD11BDA)))
    x0 = (x0 + ks[0]).astype(np.uint32)
    x1 = (x1 + ks[1]).astype(np.uint32)
    for i in range(5):
        for r in rot[(i % 2) * 4:(i % 2) * 4 + 4]:
            x0 = (x0 + x1).astype(np.uint32)
            x1 = ((x1 << np.uint32(r)) | (x1 >> np.uint32(32 - r))).astype(np.uint32)
            x1 = (x1 ^ x0).astype(np.uint32)
        x0 = (x0 + ks[(i + 1) % 3]).astype(np.uint32)
        x1 = (x1 + ks[(i + 2) % 3] + np.uint32(i + 1)).astype(np.uint32)
    return x0, x1


def _erfinv64(x):
    x = x.astype(np.float64)
    w = -np.log1p(-x * x)
    p_lo = np.full_like(w, 2.81022636e-08)
    wl = w - 2.5
    for c in (3.43273939e-07, -3.5233877e-06, -4.39150654e-06, 0.00021858087,
              -0.00125372503, -0.00417768164, 0.246640727, 1.50140941):
        p_lo = c + p_lo * wl
    ws = np.sqrt(np.maximum(w, 5.0)) - 3.0
    p_hi = np.full_like(w, -0.000200214257)
    for c in (0.000100950558, 0.00134934322, -0.00367342844, 0.00573950773,
              -0.0076224613, 0.00943887047, 1.00167406, 2.83297682):
        p_hi = c + p_hi * ws
    return np.where(w < 5.0, p_lo, p_hi) * x


def _init_state():
    n = B * P * N_DIM
    with np.errstate(over="ignore"):
        s1, s2 = _threefry2x32(0, 1, np.zeros(3, np.uint32),
                               np.arange(3, dtype=np.uint32))
        out = []
        for i in range(3):
            b1, b2 = _threefry2x32(s1[i], s2[i], np.zeros(n, np.uint32),
                                   np.arange(n, dtype=np.uint32))
            bits = (b1 ^ b2).astype(np.uint32)
            f = ((bits >> np.uint32(9)) | np.uint32(0x3F800000)).view(np.float32)
            f = f - np.float32(1.0)
            lo = np.float32(np.nextafter(np.float32(-1.0), np.float32(0.0)))
            u = np.maximum(lo, (f * (np.float32(1.0) - lo) + lo).astype(np.float32))
            v = (np.sqrt(2.0) * _erfinv64(u)).astype(np.float32)
            v = v.reshape(B, P, N_DIM)
            out.append(np.transpose(v, (1, 0, 2)) * np.float32(0.01))
    return out


_Y0, _U0, _D0 = _init_state()


def _leaky(x):
    return jnp.where(x >= 0, x, 0.01 * x)


def _graph_kernel(src_ref, dst_ref, adj_ref, lap_ref, deg_ref):
    src = src_ref[...]
    dst = dst_ref[...]
    e_iota = lax.broadcasted_iota(jnp.int32, (B, E, P), 2)
    s_oh = (src[:, :, None] == e_iota).astype(jnp.float32)
    d_oh = (dst[:, :, None] == e_iota).astype(jnp.float32)
    c = lax.dot_general(d_oh, s_oh, (((1,), (1,)), ((0,), (0,))),
                        preferred_element_type=jnp.float32)
    ct = lax.dot_general(s_oh, d_oh, (((1,), (1,)), ((0,), (0,))),
                         preferred_element_type=jnp.float32)
    deg_d = jnp.sum(c, axis=2)
    deg_s = jnp.sum(ct, axis=2)
    ii = lax.broadcasted_iota(jnp.int32, (P, P), 0)
    jj = lax.broadcasted_iota(jnp.int32, (P, P), 1)
    eye = (ii == jj).astype(jnp.float32)[None]
    dinv = lax.rsqrt(deg_d + 1.0)
    adj_ref[...] = dinv[:, :, None] * dinv[:, None, :] * (c + eye)
    lap_ref[...] = eye * (deg_s + deg_d)[:, :, None] - c - ct
    deg_ref[...] = deg_s


def _graph_ops(src, dst):
    return pl.pallas_call(
        _graph_kernel,
        out_shape=[
            jax.ShapeDtypeStruct((B, P, P), jnp.float32),
            jax.ShapeDtypeStruct((B, P, P), jnp.float32),
            jax.ShapeDtypeStruct((B, P), jnp.float32),
        ],
    )(src, dst)


def _mlp_kernel(x_ref, w_ref, b_ref, o_ref, acc_ref, *, nk, act):
    k = pl.program_id(1)

    @pl.when(k == 0)
    def _():
        acc_ref[...] = jnp.zeros_like(acc_ref)

    acc_ref[...] += jnp.dot(x_ref[...], w_ref[...],
                            preferred_element_type=jnp.float32)

    @pl.when(k == nk - 1)
    def _():
        r = acc_ref[...] + b_ref[...]
        o_ref[...] = _leaky(r) if act else r


def _mlp_layer(x, w, bias, act, kb=2048, nb=2048):
    mdim, kdim = x.shape
    ndim = w.shape[1]
    kb = min(kb, kdim)
    nk = kdim // kb
    grid = (ndim // nb, nk)
    return pl.pallas_call(
        functools.partial(_mlp_kernel, nk=nk, act=act),
        grid=grid,
        in_specs=[
            pl.BlockSpec((mdim, kb), lambda j, k: (0, k)),
            pl.BlockSpec((kb, nb), lambda j, k: (k, j)),
            pl.BlockSpec((1, nb), lambda j, k: (0, j)),
        ],
        out_specs=pl.BlockSpec((mdim, nb), lambda j, k: (0, j)),
        out_shape=jax.ShapeDtypeStruct((mdim, ndim), jnp.float32),
        scratch_shapes=[pltpu.VMEM((mdim, nb), jnp.float32)],
        compiler_params=pltpu.CompilerParams(
            dimension_semantics=("parallel", "arbitrary")),
    )(x, w, bias.reshape(1, ndim))


def _head_kernel(x_ref, adj_ref, wc1_ref, bc1_ref, wc2_ref, bc2_ref,
                 wf1_ref, bf1_ref, wf2_ref, bf2_ref, mp_ref, o_ref):
    x = x_ref[...]
    adj = adj_ref[...]
    xw = lax.dot_general(x, wc1_ref[...], (((2,), (0,)), ((), ())),
                         preferred_element_type=jnp.float32)
    h = lax.dot_general(adj, xw, (((2,), (1,)), ((0,), (0,))),
                        preferred_element_type=jnp.float32)
    h = _leaky(h + bc1_ref[...][None])
    hw = lax.dot_general(h, wc2_ref[...], (((2,), (0,)), ((), ())),
                         preferred_element_type=jnp.float32)
    h2 = lax.dot_general(adj, hw, (((2,), (1,)), ((0,), (0,))),
                         preferred_element_type=jnp.float32)
    h2 = _leaky(h2 + bc2_ref[...][None])
    pool = jnp.mean(h2, axis=1)
    f = _leaky(jnp.dot(pool, wf1_ref[...],
                       preferred_element_type=jnp.float32) + bf1_ref[...])
    g = jnp.dot(f, wf2_ref[...],
                preferred_element_type=jnp.float32) + bf2_ref[...]
    mp = mp_ref[...]
    acc = jnp.zeros((B, P * 4), jnp.float32)
    for k in range(K_IT):
        acc = acc + g[:, k * P * 4:(k + 1) * P * 4]
        o_ref[:, k * P * 4:(k + 1) * P * 4] = jax.nn.sigmoid(acc) * mp


def _head(x, adj, wc1, bc1, wc2, bc2, wf1, bf1, wf2, bf2, mp):
    return pl.pallas_call(
        _head_kernel,
        out_shape=jax.ShapeDtypeStruct((B, K_IT * P * 4), jnp.float32),
    )(x, adj, wc1, bc1.reshape(1, -1), wc2, bc2.reshape(1, -1),
      wf1, bf1.reshape(1, -1), wf2, bf2.reshape(1, -1), mp)


def _admm_kernel(a0_ref, bt_ref, y0_ref, u0_ref, d0_ref, lap_ref,
                 ha_ref, ht_ref, hr_ref, he_ref, sn_ref, o_ref,
                 atb_ref, y_ref, u_ref, d_ref):
    a0 = a0_ref[...]
    atb_ref[...] = lax.dot_general(bt_ref[...], a0,
                                   (((2,), (1,)), ((0,), (0,))),
                                   preferred_element_type=jnp.float32)
    y_ref[...] = y0_ref[...]
    u_ref[...] = u0_ref[...]
    d_ref[...] = d0_ref[...]
    sn = sn_ref[...][:, :, None]

    def step(k, _):
        al = jnp.reshape(ha_ref[pl.ds(k, 1)], (P, B))[:, :, None]
        ta = jnp.reshape(ht_ref[pl.ds(k, 1)], (P, B))[:, :, None]
        rh = jnp.reshape(hr_ref[pl.ds(k, 1)], (P, B))[:, :, None]
        et = jnp.reshape(he_ref[pl.ds(k, 1)], (P, B))[:, :, None]
        y = y_ref[...]
        ay = lax.dot_general(y, a0, (((2,), (2,)), ((0,), (0,))),
                             preferred_element_type=jnp.float32)
        atay = lax.dot_general(ay, a0, (((2,), (1,)), ((0,), (0,))),
                               preferred_element_type=jnp.float32)
        grad = (atay - atb_ref[...] + jnp.sign(y) * ta
                + u_ref[...] * sn + d_ref[...] * rh)
        y_next = y - al * grad
        for bb in range(B):
            yb = y_next[:, bb, :]
            lb = lap_ref[bb]
            db = jnp.dot(lb, yb, preferred_element_type=jnp.float32)
            d_ref[:, bb, :] = db
            o_ref[pl.ds(k, 1), bb] = yb[None]
        u_ref[...] = u_ref[...] + d_ref[...] * et
        y_ref[...] = y_next
        return 0

    lax.fori_loop(0, K_IT, step, 0)


def _admm(a0, bt, y0, u0, d0, lap, ha, ht, hr, he, sn):
    return pl.pallas_call(
        _admm_kernel,
        out_shape=jax.ShapeDtypeStruct((K_IT, B, P, N_DIM), jnp.float32),
        scratch_shapes=[
            pltpu.VMEM((P, B, N_DIM), jnp.float32),
            pltpu.VMEM((P, B, N_DIM), jnp.float32),
            pltpu.VMEM((P, B, N_DIM), jnp.float32),
            pltpu.VMEM((P, B, N_DIM), jnp.float32),
        ],
    )(a0, bt, y0, u0, d0, lap, ha, ht, hr, he, sn)


def kernel(b, A, W1, b1, W2, b2, W3, b3, Wc1, bc1, Wc2, bc2,
           Wf1, bf1, Wf2, bf2, max_param, edge_index):
    src = edge_index[:, 0, :].astype(jnp.int32)
    dst = edge_index[:, 1, :].astype(jnp.int32)
    adj, lap, deg = _graph_ops(src, dst)

    x0 = b.reshape(B, P * M)
    x1 = _mlp_layer(x0, W1, b1, act=True)
    x2 = _mlp_layer(x1, W2, b2, act=True)
    x3 = _mlp_layer(x2, W3, b3, act=False)

    mp = jnp.tile(max_param.reshape(-1), P).reshape(1, P * 4)
    hyp_flat = _head(x3.reshape(B, P, 4 * H), adj, Wc1, bc1, Wc2, bc2,
                     Wf1, bf1, Wf2, bf2, mp)
    hyp4 = hyp_flat.reshape(B, K_IT, P, 4)
    ha = jnp.transpose(hyp4[..., 0], (1, 2, 0))
    ht = jnp.transpose(hyp4[..., 1], (1, 2, 0))
    hr = jnp.transpose(hyp4[..., 2], (1, 2, 0))
    he = jnp.transpose(hyp4[..., 3], (1, 2, 0))

    a0 = A[0]
    bt = jnp.transpose(b[..., 0], (1, 0, 2))
    snt = jnp.transpose(deg, (1, 0))

    ys = _admm(a0, bt, jnp.asarray(_Y0), jnp.asarray(_U0), jnp.asarray(_D0),
               lap, ha, ht, hr, he, snt)
    return ys[..., None]


# speedup vs baseline: 14.8613x; 1.0336x over previous
import functools

import jax
import jax.numpy as jnp
import numpy as np
from jax import lax
from jax.experimental import pallas as pl
from jax.experimental.pallas import tpu as pltpu

B = 16
P = 64
M = 32
N_DIM = 256
H = 64
K_IT = 10
E = 512


def _threefry2x32(k0, k1, x0, x1):
    rot = (13, 15, 26, 6, 17, 29, 16, 24)
    k0 = np.uint32(k0)
    k1 = np.uint32(k1)
    ks = (k0, k1, np.uint32(k0 ^ k1 ^ np.uint32(0x1BD11BDA)))
    x0 = (x0 + ks[0]).astype(np.uint32)
    x1 = (x1 + ks[1]).astype(np.uint32)
    for i in range(5):
        for r in rot[(i % 2) * 4:(i % 2) * 4 + 4]:
            x0 = (x0 + x1).astype(np.uint32)
            x1 = ((x1 << np.uint32(r)) | (x1 >> np.uint32(32 - r))).astype(np.uint32)
            x1 = (x1 ^ x0).astype(np.uint32)
        x0 = (x0 + ks[(i + 1) % 3]).astype(np.uint32)
        x1 = (x1 + ks[(i + 2) % 3] + np.uint32(i + 1)).astype(np.uint32)
    return x0, x1


def _erfinv64(x):
    x = x.astype(np.float64)
    w = -np.log1p(-x * x)
    p_lo = np.full_like(w, 2.81022636e-08)
    wl = w - 2.5
    for c in (3.43273939e-07, -3.5233877e-06, -4.39150654e-06, 0.00021858087,
              -0.00125372503, -0.00417768164, 0.246640727, 1.50140941):
        p_lo = c + p_lo * wl
    ws = np.sqrt(np.maximum(w, 5.0)) - 3.0
    p_hi = np.full_like(w, -0.000200214257)
    for c in (0.000100950558, 0.00134934322, -0.00367342844, 0.00573950773,
              -0.0076224613, 0.00943887047, 1.00167406, 2.83297682):
        p_hi = c + p_hi * ws
    return np.where(w < 5.0, p_lo, p_hi) * x


def _init_state():
    n = B * P * N_DIM
    with np.errstate(over="ignore"):
        s1, s2 = _threefry2x32(0, 1, np.zeros(3, np.uint32),
                               np.arange(3, dtype=np.uint32))
        out = []
        for i in range(3):
            b1, b2 = _threefry2x32(s1[i], s2[i], np.zeros(n, np.uint32),
                                   np.arange(n, dtype=np.uint32))
            bits = (b1 ^ b2).astype(np.uint32)
            f = ((bits >> np.uint32(9)) | np.uint32(0x3F800000)).view(np.float32)
            f = f - np.float32(1.0)
            lo = np.float32(np.nextafter(np.float32(-1.0), np.float32(0.0)))
            u = np.maximum(lo, (f * (np.float32(1.0) - lo) + lo).astype(np.float32))
            v = (np.sqrt(2.0) * _erfinv64(u)).astype(np.float32)
            v = v.reshape(B, P, N_DIM)
            out.append(np.transpose(v, (1, 0, 2)) * np.float32(0.01))
    return out


_Y0, _U0, _D0 = _init_state()


def _leaky(x):
    return jnp.where(x >= 0, x, 0.01 * x)


def _mlp_kernel(x_ref, w_ref, b_ref, o_ref, acc_ref, *, nk, act):
    k = pl.program_id(1)

    @pl.when(k == 0)
    def _():
        acc_ref[...] = jnp.zeros_like(acc_ref)

    acc_ref[...] += jnp.dot(x_ref[...], w_ref[...],
                            preferred_element_type=jnp.float32)

    @pl.when(k == nk - 1)
    def _():
        r = acc_ref[...] + b_ref[...]
        o_ref[...] = _leaky(r) if act else r


def _mlp_layer(x, w, bias, act, kb=2048, nb=2048):
    mdim, kdim = x.shape
    ndim = w.shape[1]
    kb = min(kb, kdim)
    nk = kdim // kb
    grid = (ndim // nb, nk)
    return pl.pallas_call(
        functools.partial(_mlp_kernel, nk=nk, act=act),
        grid=grid,
        in_specs=[
            pl.BlockSpec((mdim, kb), lambda j, k: (0, k)),
            pl.BlockSpec((kb, nb), lambda j, k: (k, j)),
            pl.BlockSpec((1, nb), lambda j, k: (0, j)),
        ],
        out_specs=pl.BlockSpec((mdim, nb), lambda j, k: (0, j)),
        out_shape=jax.ShapeDtypeStruct((mdim, ndim), jnp.float32),
        scratch_shapes=[pltpu.VMEM((mdim, nb), jnp.float32)],
        compiler_params=pltpu.CompilerParams(
            dimension_semantics=("parallel", "arbitrary")),
    )(x, w, bias.reshape(1, ndim))


def _mega_kernel(edge_ref, x_ref, wc1_ref, bc1_ref, wc2_ref, bc2_ref,
                 wf1_ref, bf1_ref, wf2_ref, bf2_ref, mp_ref,
                 a0_ref, bt_ref, y0_ref, u0_ref, d0_ref, o_ref,
                 atb_ref, y_ref, u_ref, d_ref,
                 ha_ref, ht_ref, hr_ref, he_ref):
    f32 = jnp.float32
    src = edge_ref[:, 0, :]
    dst = edge_ref[:, 1, :]
    e_iota = lax.broadcasted_iota(jnp.int32, (B, E, P), 2)
    s_oh = (src[:, :, None] == e_iota).astype(f32)
    d_oh = (dst[:, :, None] == e_iota).astype(f32)
    c = lax.dot_general(d_oh, s_oh, (((1,), (1,)), ((0,), (0,))),
                        preferred_element_type=f32)
    ct = lax.dot_general(s_oh, d_oh, (((1,), (1,)), ((0,), (0,))),
                         preferred_element_type=f32)
    deg_d = jnp.sum(c, axis=2)
    deg_s = jnp.sum(ct, axis=2)
    ii = lax.broadcasted_iota(jnp.int32, (P, P), 0)
    jj = lax.broadcasted_iota(jnp.int32, (P, P), 1)
    eye = (ii == jj).astype(f32)
    dinv = lax.rsqrt(deg_d + 1.0)
    adj = dinv[:, :, None] * dinv[:, None, :] * (c + eye[None])
    lap = eye[None] * (deg_s + deg_d)[:, :, None] - c - ct
    sn = lax.dot_general(eye, deg_s, (((1,), (1,)), ((), ())),
                         preferred_element_type=f32)[:, :, None]

    x = jnp.reshape(x_ref[...], (B, P, 4 * H))
    xw = lax.dot_general(x, wc1_ref[...], (((2,), (0,)), ((), ())),
                         preferred_element_type=f32)
    h = lax.dot_general(adj, xw, (((2,), (1,)), ((0,), (0,))),
                        preferred_element_type=f32)
    h = _leaky(h + bc1_ref[...][None])
    hw = lax.dot_general(h, wc2_ref[...], (((2,), (0,)), ((), ())),
                         preferred_element_type=f32)
    h2 = lax.dot_general(adj, hw, (((2,), (1,)), ((0,), (0,))),
                         preferred_element_type=f32)
    h2 = _leaky(h2 + bc2_ref[...][None])
    pool = jnp.mean(h2, axis=1)
    f = _leaky(jnp.dot(pool, wf1_ref[...],
                       preferred_element_type=f32) + bf1_ref[...])
    g = jnp.dot(f, wf2_ref[...],
                preferred_element_type=f32) + bf2_ref[...]
    mp = mp_ref[...]

    qq = lax.broadcasted_iota(jnp.int32, (P * 4, P), 0)
    pp = lax.broadcasted_iota(jnp.int32, (P * 4, P), 1)
    refs = (ha_ref, ht_ref, hr_ref, he_ref)
    acc = jnp.zeros((B, P * 4), f32)
    for k in range(K_IT):
        acc = acc + g[:, k * P * 4:(k + 1) * P * 4]
        hyp_k = jax.nn.sigmoid(acc) * mp
        for j in range(4):
            sel = (qq == 4 * pp + j).astype(f32)
            refs[j][k] = lax.dot_general(sel, hyp_k, (((0,), (1,)), ((), ())),
                                         preferred_element_type=f32)

    a0 = a0_ref[...]
    atb_ref[...] = lax.dot_general(bt_ref[...], a0,
                                   (((2,), (1,)), ((0,), (0,))),
                                   preferred_element_type=f32)
    y_ref[...] = y0_ref[...]
    u_ref[...] = u0_ref[...]
    d_ref[...] = d0_ref[...]

    def step(k, _):
        al = jnp.reshape(ha_ref[pl.ds(k, 1)], (P, B))[:, :, None]
        ta = jnp.reshape(ht_ref[pl.ds(k, 1)], (P, B))[:, :, None]
        rh = jnp.reshape(hr_ref[pl.ds(k, 1)], (P, B))[:, :, None]
        et = jnp.reshape(he_ref[pl.ds(k, 1)], (P, B))[:, :, None]
        y = y_ref[...]
        ay = lax.dot_general(y, a0, (((2,), (2,)), ((0,), (0,))),
                             preferred_element_type=f32)
        atay = lax.dot_general(ay, a0, (((2,), (1,)), ((0,), (0,))),
                               preferred_element_type=f32)
        grad = (atay - atb_ref[...] + jnp.sign(y) * ta
                + u_ref[...] * sn + d_ref[...] * rh)
        y_next = y - al * grad
        for bb in range(B):
            yb = y_next[:, bb, :]
            db = jnp.dot(lap[bb], yb, preferred_element_type=f32)
            d_ref[:, bb, :] = db
            o_ref[pl.ds(k, 1), bb] = yb[None]
        u_ref[...] = u_ref[...] + d_ref[...] * et
        y_ref[...] = y_next
        return 0

    lax.fori_loop(0, K_IT, step, 0)


def _mega(edge, x3, wc1, bc1, wc2, bc2, wf1, bf1, wf2, bf2, mp,
          a0, bt, y0, u0, d0):
    return pl.pallas_call(
        _mega_kernel,
        out_shape=jax.ShapeDtypeStruct((K_IT, B, P, N_DIM), jnp.float32),
        scratch_shapes=[
            pltpu.VMEM((P, B, N_DIM), jnp.float32),
            pltpu.VMEM((P, B, N_DIM), jnp.float32),
            pltpu.VMEM((P, B, N_DIM), jnp.float32),
            pltpu.VMEM((P, B, N_DIM), jnp.float32),
            pltpu.VMEM((K_IT, P, B), jnp.float32),
            pltpu.VMEM((K_IT, P, B), jnp.float32),
            pltpu.VMEM((K_IT, P, B), jnp.float32),
            pltpu.VMEM((K_IT, P, B), jnp.float32),
        ],
    )(edge, x3, wc1, bc1.reshape(1, -1), wc2, bc2.reshape(1, -1),
      wf1, bf1.reshape(1, -1), wf2, bf2.reshape(1, -1), mp,
      a0, bt, y0, u0, d0)


def kernel(b, A, W1, b1, W2, b2, W3, b3, Wc1, bc1, Wc2, bc2,
           Wf1, bf1, Wf2, bf2, max_param, edge_index):
    edge = edge_index.astype(jnp.int32)

    x0 = b.reshape(B, P * M)
    x1 = _mlp_layer(x0, W1, b1, act=True)
    x2 = _mlp_layer(x1, W2, b2, act=True)
    x3 = _mlp_layer(x2, W3, b3, act=False)

    mp = jnp.tile(max_param.reshape(-1), P).reshape(1, P * 4)
    a0 = A[0]
    bt = jnp.transpose(b[..., 0], (1, 0, 2))

    ys = _mega(edge, x3, Wc1, bc1, Wc2, bc2, Wf1, bf1, Wf2, bf2, mp,
               a0, bt, jnp.asarray(_Y0), jnp.asarray(_U0), jnp.asarray(_D0))
    return ys[..., None]


# speedup vs baseline: 14.8725x; 1.0008x over previous
import functools

import jax
import jax.numpy as jnp
import numpy as np
from jax import lax
from jax.experimental import pallas as pl
from jax.experimental.pallas import tpu as pltpu
from jax.experimental.pallas import tpu_sc as plsc

B = 16
P = 64
M = 32
N_DIM = 256
H = 64
K_IT = 10
E = 512


def _threefry2x32(k0, k1, x0, x1):
    rot = (13, 15, 26, 6, 17, 29, 16, 24)
    k0 = np.uint32(k0)
    k1 = np.uint32(k1)
    ks = (k0, k1, np.uint32(k0 ^ k1 ^ np.uint32(0x1BD11BDA)))
    x0 = (x0 + ks[0]).astype(np.uint32)
    x1 = (x1 + ks[1]).astype(np.uint32)
    for i in range(5):
        for r in rot[(i % 2) * 4:(i % 2) * 4 + 4]:
            x0 = (x0 + x1).astype(np.uint32)
            x1 = ((x1 << np.uint32(r)) | (x1 >> np.uint32(32 - r))).astype(np.uint32)
            x1 = (x1 ^ x0).astype(np.uint32)
        x0 = (x0 + ks[(i + 1) % 3]).astype(np.uint32)
        x1 = (x1 + ks[(i + 2) % 3] + np.uint32(i + 1)).astype(np.uint32)
    return x0, x1


def _erfinv64(x):
    x = x.astype(np.float64)
    w = -np.log1p(-x * x)
    p_lo = np.full_like(w, 2.81022636e-08)
    wl = w - 2.5
    for c in (3.43273939e-07, -3.5233877e-06, -4.39150654e-06, 0.00021858087,
              -0.00125372503, -0.00417768164, 0.246640727, 1.50140941):
        p_lo = c + p_lo * wl
    ws = np.sqrt(np.maximum(w, 5.0)) - 3.0
    p_hi = np.full_like(w, -0.000200214257)
    for c in (0.000100950558, 0.00134934322, -0.00367342844, 0.00573950773,
              -0.0076224613, 0.00943887047, 1.00167406, 2.83297682):
        p_hi = c + p_hi * ws
    return np.where(w < 5.0, p_lo, p_hi) * x


def _init_state():
    n = B * P * N_DIM
    with np.errstate(over="ignore"):
        s1, s2 = _threefry2x32(0, 1, np.zeros(3, np.uint32),
                               np.arange(3, dtype=np.uint32))
        out = []
        for i in range(3):
            b1, b2 = _threefry2x32(s1[i], s2[i], np.zeros(n, np.uint32),
                                   np.arange(n, dtype=np.uint32))
            bits = (b1 ^ b2).astype(np.uint32)
            f = ((bits >> np.uint32(9)) | np.uint32(0x3F800000)).view(np.float32)
            f = f - np.float32(1.0)
            lo = np.float32(np.nextafter(np.float32(-1.0), np.float32(0.0)))
            u = np.maximum(lo, (f * (np.float32(1.0) - lo) + lo).astype(np.float32))
            v = (np.sqrt(2.0) * _erfinv64(u)).astype(np.float32)
            v = v.reshape(B, P, N_DIM)
            out.append(np.transpose(v, (1, 0, 2)) * np.float32(0.01))
    return out


_Y0, _U0, _D0 = _init_state()


def _leaky(x):
    return jnp.where(x >= 0, x, 0.01 * x)


def _sc_edge_body(edge_hbm, c_hbm, src_v, dst_v, cnt_v):
    cid = lax.axis_index("c")
    sid = lax.axis_index("s")

    @pl.when(cid == 0)
    def _():
        bb = sid
        pltpu.sync_copy(edge_hbm.at[bb, 0], src_v)
        pltpu.sync_copy(edge_hbm.at[bb, 1], dst_v)
        zeros16 = jnp.zeros((16,), jnp.float32)

        def zbody(i, carry):
            cnt_v[pl.ds(i * 16, 16)] = zeros16
            return carry

        lax.fori_loop(0, P * P // 16, zbody, 0)
        lanes = lax.iota(jnp.int32, 16)
        ones16 = jnp.ones((16,), jnp.float32)
        for ch in range(E // 16):
            s = src_v[pl.ds(ch * 16, 16)]
            d = dst_v[pl.ds(ch * 16, 16)]
            flat = d * P + s
            for l in range(16):
                plsc.addupdate_scatter(cnt_v, [flat], ones16,
                                       mask=lanes == l)
        pltpu.sync_copy(cnt_v, c_hbm.at[bb])


def _sc_edge_counts(edge):
    mesh = plsc.VectorSubcoreMesh(core_axis_name="c", subcore_axis_name="s")
    fn = functools.partial(
        pl.kernel,
        mesh=mesh,
        out_type=jax.ShapeDtypeStruct((B, P * P), jnp.float32),
        scratch_types=[
            pltpu.VMEM((E,), jnp.int32),
            pltpu.VMEM((E,), jnp.int32),
            pltpu.VMEM((P * P,), jnp.float32),
        ],
        compiler_params=pltpu.CompilerParams(needs_layout_passes=False),
    )(_sc_edge_body)
    return fn(edge)


def _mlp_kernel(x_ref, w_ref, b_ref, o_ref, acc_ref, *, nk, act):
    k = pl.program_id(1)

    @pl.when(k == 0)
    def _():
        acc_ref[...] = jnp.zeros_like(acc_ref)

    acc_ref[...] += jnp.dot(x_ref[...], w_ref[...],
                            preferred_element_type=jnp.float32)

    @pl.when(k == nk - 1)
    def _():
        r = acc_ref[...] + b_ref[...]
        o_ref[...] = _leaky(r) if act else r


def _mlp_layer(x, w, bias, act, kb=2048, nb=2048):
    mdim, kdim = x.shape
    ndim = w.shape[1]
    kb = min(kb, kdim)
    nk = kdim // kb
    grid = (ndim // nb, nk)
    return pl.pallas_call(
        functools.partial(_mlp_kernel, nk=nk, act=act),
        grid=grid,
        in_specs=[
            pl.BlockSpec((mdim, kb), lambda j, k: (0, k)),
            pl.BlockSpec((kb, nb), lambda j, k: (k, j)),
            pl.BlockSpec((1, nb), lambda j, k: (0, j)),
        ],
        out_specs=pl.BlockSpec((mdim, nb), lambda j, k: (0, j)),
        out_shape=jax.ShapeDtypeStruct((mdim, ndim), jnp.float32),
        scratch_shapes=[pltpu.VMEM((mdim, nb), jnp.float32)],
        compiler_params=pltpu.CompilerParams(
            dimension_semantics=("parallel", "arbitrary")),
    )(x, w, bias.reshape(1, ndim))


def _mega_kernel(c_ref, x_ref, wc1_ref, bc1_ref, wc2_ref, bc2_ref,
                 wf1_ref, bf1_ref, wf2_ref, bf2_ref, mp_ref,
                 a0_ref, bt_ref, y0_ref, u0_ref, d0_ref, o_ref,
                 atb_ref, y_ref, u_ref, d_ref,
                 ha_ref, ht_ref, hr_ref, he_ref):
    f32 = jnp.float32
    c = jnp.reshape(c_ref[...], (B, P, P))
    ii = lax.broadcasted_iota(jnp.int32, (P, P), 0)
    jj = lax.broadcasted_iota(jnp.int32, (P, P), 1)
    eye = (ii == jj).astype(f32)
    ct = lax.dot_general(c, eye, (((1,), (0,)), ((), ())),
                         preferred_element_type=f32)
    deg_d = jnp.sum(c, axis=2)
    deg_s = jnp.sum(ct, axis=2)
    dinv = lax.rsqrt(deg_d + 1.0)
    adj = dinv[:, :, None] * dinv[:, None, :] * (c + eye[None])
    lap = eye[None] * (deg_s + deg_d)[:, :, None] - c - ct
    sn = lax.dot_general(eye, deg_s, (((1,), (1,)), ((), ())),
                         preferred_element_type=f32)[:, :, None]

    x = jnp.reshape(x_ref[...], (B, P, 4 * H))
    xw = lax.dot_general(x, wc1_ref[...], (((2,), (0,)), ((), ())),
                         preferred_element_type=f32)
    h = lax.dot_general(adj, xw, (((2,), (1,)), ((0,), (0,))),
                        preferred_element_type=f32)
    h = _leaky(h + bc1_ref[...][None])
    hw = lax.dot_general(h, wc2_ref[...], (((2,), (0,)), ((), ())),
                         preferred_element_type=f32)
    h2 = lax.dot_general(adj, hw, (((2,), (1,)), ((0,), (0,))),
                         preferred_element_type=f32)
    h2 = _leaky(h2 + bc2_ref[...][None])
    pool = jnp.mean(h2, axis=1)
    f = _leaky(jnp.dot(pool, wf1_ref[...],
                       preferred_element_type=f32) + bf1_ref[...])
    g = jnp.dot(f, wf2_ref[...],
                preferred_element_type=f32) + bf2_ref[...]
    mp = mp_ref[...]

    qq = lax.broadcasted_iota(jnp.int32, (P * 4, P), 0)
    pp = lax.broadcasted_iota(jnp.int32, (P * 4, P), 1)
    refs = (ha_ref, ht_ref, hr_ref, he_ref)
    acc = jnp.zeros((B, P * 4), f32)
    for k in range(K_IT):
        acc = acc + g[:, k * P * 4:(k + 1) * P * 4]
        hyp_k = jax.nn.sigmoid(acc) * mp
        for j in range(4):
            sel = (qq == 4 * pp + j).astype(f32)
            refs[j][k] = lax.dot_general(sel, hyp_k, (((0,), (1,)), ((), ())),
                                         preferred_element_type=f32)

    a0 = a0_ref[...]
    atb_ref[...] = lax.dot_general(bt_ref[...], a0,
                                   (((2,), (1,)), ((0,), (0,))),
                                   preferred_element_type=f32)
    y_ref[...] = y0_ref[...]
    u_ref[...] = u0_ref[...]
    d_ref[...] = d0_ref[...]

    def step(k, _):
        al = jnp.reshape(ha_ref[pl.ds(k, 1)], (P, B))[:, :, None]
        ta = jnp.reshape(ht_ref[pl.ds(k, 1)], (P, B))[:, :, None]
        rh = jnp.reshape(hr_ref[pl.ds(k, 1)], (P, B))[:, :, None]
        et = jnp.reshape(he_ref[pl.ds(k, 1)], (P, B))[:, :, None]
        y = y_ref[...]
        ay = lax.dot_general(y, a0, (((2,), (2,)), ((0,), (0,))),
                             preferred_element_type=f32)
        atay = lax.dot_general(ay, a0, (((2,), (1,)), ((0,), (0,))),
                               preferred_element_type=f32)
        grad = (atay - atb_ref[...] + jnp.sign(y) * ta
                + u_ref[...] * sn + d_ref[...] * rh)
        y_next = y - al * grad
        for bb in range(B):
            yb = y_next[:, bb, :]
            db = jnp.dot(lap[bb], yb, preferred_element_type=f32)
            d_ref[:, bb, :] = db
            o_ref[pl.ds(k, 1), bb] = yb[None]
        u_ref[...] = u_ref[...] + d_ref[...] * et
        y_ref[...] = y_next
        return 0

    lax.fori_loop(0, K_IT, step, 0)


def _mega(c4, x3, wc1, bc1, wc2, bc2, wf1, bf1, wf2, bf2, mp,
          a0, bt, y0, u0, d0):
    return pl.pallas_call(
        _mega_kernel,
        out_shape=jax.ShapeDtypeStruct((K_IT, B, P, N_DIM), jnp.float32),
        scratch_shapes=[
            pltpu.VMEM((P, B, N_DIM), jnp.float32),
            pltpu.VMEM((P, B, N_DIM), jnp.float32),
            pltpu.VMEM((P, B, N_DIM), jnp.float32),
            pltpu.VMEM((P, B, N_DIM), jnp.float32),
            pltpu.VMEM((K_IT, P, B), jnp.float32),
            pltpu.VMEM((K_IT, P, B), jnp.float32),
            pltpu.VMEM((K_IT, P, B), jnp.float32),
            pltpu.VMEM((K_IT, P, B), jnp.float32),
        ],
    )(c4, x3, wc1, bc1.reshape(1, -1), wc2, bc2.reshape(1, -1),
      wf1, bf1.reshape(1, -1), wf2, bf2.reshape(1, -1), mp,
      a0, bt, y0, u0, d0)


def kernel(b, A, W1, b1, W2, b2, W3, b3, Wc1, bc1, Wc2, bc2,
           Wf1, bf1, Wf2, bf2, max_param, edge_index):
    edge = edge_index.astype(jnp.int32)
    c4 = _sc_edge_counts(edge)

    x0 = b.reshape(B, P * M)
    x1 = _mlp_layer(x0, W1, b1, act=True)
    x2 = _mlp_layer(x1, W2, b2, act=True)
    x3 = _mlp_layer(x2, W3, b3, act=False)

    mp = jnp.tile(max_param.reshape(-1), P).reshape(1, P * 4)
    a0 = A[0]
    bt = jnp.transpose(b[..., 0], (1, 0, 2))

    ys = _mega(c4, x3, Wc1, bc1, Wc2, bc2, Wf1, bf1, Wf2, bf2, mp,
               a0, bt, jnp.asarray(_Y0), jnp.asarray(_U0), jnp.asarray(_D0))
    return ys[..., None]
